# Initial kernel scaffold; baseline (speedup 1.0000x reference)
import jax
import jax.numpy as jnp
from jax.experimental import pallas as pl


def kernel(q, k, v):
    raise NotImplementedError("write your pallas kernel here")


# speedup vs baseline: 5.5226x; 5.5226x over previous
import functools

import jax
import jax.numpy as jnp
from jax import lax
from jax.experimental import pallas as pl
from jax.experimental.pallas import tpu as pltpu
from jax.experimental.pallas import tpu_sc as plsc

N = 4096
C = 64
B = 4
KK = 8
QB = 256
D = B * C
NC, NS = 2, 16
NW = NC * NS
PPW = N // NW
PCH = 16
NCH = PPW // PCH


def _topk_body(qb_ref, kf_ref, idx_ref, w_ref):
    qb = qb_ref[...]
    kf = kf_ref[...]
    s = lax.dot_general(qb, kf, (((0,), (0,)), ((), ())),
                        preferred_element_type=jnp.float32)
    kn = jnp.sum(kf * kf, axis=0)
    cur = kn[None, :] - 2.0 * s
    iota = lax.broadcasted_iota(jnp.int32, (QB, N), 1)
    vals, idxs = [], []
    for j in range(KK):
        m = jnp.min(cur, axis=1, keepdims=True)
        im = jnp.where(cur <= m, iota, jnp.int32(N))
        mi = jnp.min(im, axis=1, keepdims=True)
        vals.append(m)
        idxs.append(mi)
        if j + 1 < KK:
            cur = jnp.where(iota == mi, jnp.float32(3.0e38), cur)
    es = [jnp.exp(vals[0] - vj) for vj in vals]
    den = es[0]
    for e in es[1:]:
        den = den + e
    for j in range(KK):
        idx_ref[j, :] = idxs[j][:, 0]
        w_ref[j, :] = (es[j] / den)[:, 0]


def _topk_call(qf, kf):
    return pl.pallas_call(
        _topk_body,
        grid=(N // QB,),
        in_specs=[pl.BlockSpec((C, QB), lambda i: (0, i)),
                  pl.BlockSpec((C, N), lambda i: (0, 0))],
        out_specs=[pl.BlockSpec((KK, QB), lambda i: (0, i)),
                   pl.BlockSpec((KK, QB), lambda i: (0, i))],
        out_shape=[jax.ShapeDtypeStruct((KK, N), jnp.int32),
                   jax.ShapeDtypeStruct((KK, N), jnp.float32)],
    )(qf, kf)


@functools.cache
def _make_sc_combine():
    mesh = plsc.VectorSubcoreMesh(core_axis_name="c", subcore_axis_name="s",
                                  num_cores=NC, num_subcores=NS)

    @functools.partial(
        pl.kernel,
        out_type=jax.ShapeDtypeStruct((N, D), jnp.float32),
        mesh=mesh,
        scratch_types=[
            pltpu.VMEM((KK, PPW), jnp.int32),
            pltpu.VMEM((KK, PPW), jnp.float32),
            pltpu.VMEM((KK, PCH, D), jnp.float32),
            pltpu.VMEM((PCH, D), jnp.float32),
            pltpu.SemaphoreType.DMA,
        ],
        compiler_params=pltpu.CompilerParams(use_tc_tiling_on_sc=False,
                                             needs_layout_passes=False),
    )
    def sc_combine(vt, idxm, wm, out, idx_v, w_v, rows_v, ob_v, sem):
        wid = lax.axis_index("s") * NC + lax.axis_index("c")
        base = wid * PPW
        for j in range(KK):
            pltpu.sync_copy(idxm.at[j, pl.ds(base, PPW)], idx_v.at[j])
            pltpu.sync_copy(wm.at[j, pl.ds(base, PPW)], w_v.at[j])
        lanes = lax.iota(jnp.int32, PCH)
        jvecs = [jnp.full((PCH,), j, jnp.int32) for j in range(KK)]

        @pl.loop(0, NCH)
        def _chunk(ci):
            p0 = ci * PCH
            descs = [
                pltpu.async_copy(vt.at[idx_v[j, pl.ds(p0, PCH)]],
                                 rows_v.at[j], sem)
                for j in range(KK)
            ]
            for dsc in descs:
                dsc.wait()
            wv = [w_v[j, pl.ds(p0, PCH)] for j in range(KK)]

            @pl.loop(0, D)
            def _chan(cc):
                cv = jnp.full((PCH,), cc, jnp.int32)
                acc = wv[0] * plsc.load_gather(rows_v, [jvecs[0], lanes, cv])
                for j in range(1, KK):
                    acc = acc + wv[j] * plsc.load_gather(
                        rows_v, [jvecs[j], lanes, cv])
                plsc.store_scatter(ob_v, [lanes, cv], acc)

            pltpu.sync_copy(ob_v, out.at[pl.ds(base + p0, PCH)])

    return sc_combine


def kernel(q, k, v):
    qf = q[0].reshape(C, N)
    kf = k[0].reshape(C, N)
    idxm, wm = _topk_call(qf, kf)
    vt = v.reshape(B, C, N).transpose(2, 0, 1).reshape(N, D)
    out = _make_sc_combine()(vt, idxm, wm)
    return out.reshape(64, 64, B, C).transpose(2, 3, 0, 1)


# speedup vs baseline: 9.0252x; 1.6342x over previous
import functools

import jax
import jax.numpy as jnp
from jax import lax
from jax.experimental import pallas as pl
from jax.experimental.pallas import tpu as pltpu
from jax.experimental.pallas import tpu_sc as plsc

N = 4096
C = 64
B = 4
KK = 8
QB = 256
D = B * C
NC, NS = 2, 16
NW = NC * NS
PPW = N // NW
PCH = 16
NCH = PPW // PCH


def _topk_body(qb_ref, kf_ref, idx_ref, w_ref):
    qb = qb_ref[...]
    kf = kf_ref[...]
    s = lax.dot_general(qb, kf, (((0,), (0,)), ((), ())),
                        preferred_element_type=jnp.float32)
    kn = jnp.sum(kf * kf, axis=0)
    cur = kn[None, :] - 2.0 * s
    iota = lax.broadcasted_iota(jnp.int32, (QB, N), 1)
    vals, idxs = [], []
    for j in range(KK):
        m = jnp.min(cur, axis=1, keepdims=True)
        im = jnp.where(cur <= m, iota, jnp.int32(N))
        mi = jnp.min(im, axis=1, keepdims=True)
        vals.append(m)
        idxs.append(mi)
        if j + 1 < KK:
            cur = jnp.where(iota == mi, jnp.float32(3.0e38), cur)
    es = [jnp.exp(vals[0] - vj) for vj in vals]
    den = es[0]
    for e in es[1:]:
        den = den + e
    for j in range(KK):
        idx_ref[j, :] = idxs[j][:, 0]
        w_ref[j, :] = (es[j] / den)[:, 0]


def _topk_call(qf, kf):
    return pl.pallas_call(
        _topk_body,
        grid=(N // QB,),
        in_specs=[pl.BlockSpec((C, QB), lambda i: (0, i)),
                  pl.BlockSpec((C, N), lambda i: (0, 0))],
        out_specs=[pl.BlockSpec((KK, QB), lambda i: (0, i)),
                   pl.BlockSpec((KK, QB), lambda i: (0, i))],
        out_shape=[jax.ShapeDtypeStruct((KK, N), jnp.int32),
                   jax.ShapeDtypeStruct((KK, N), jnp.float32)],
    )(qf, kf)


def _bcast_lane(vec, idx):
    dn = lax.GatherDimensionNumbers(offset_dims=(), collapsed_slice_dims=(0,),
                                    start_index_map=(0,))
    return lax.gather(vec, idx, dn, (1,),
                      mode=lax.GatherScatterMode.PROMISE_IN_BOUNDS)


@functools.cache
def _make_sc_combine():
    mesh = plsc.VectorSubcoreMesh(core_axis_name="c", subcore_axis_name="s",
                                  num_cores=NC, num_subcores=NS)

    @functools.partial(
        pl.kernel,
        out_type=jax.ShapeDtypeStruct((N, D), jnp.float32),
        mesh=mesh,
        scratch_types=[
            pltpu.VMEM((KK, PPW), jnp.int32),
            pltpu.VMEM((KK, PPW), jnp.float32),
            pltpu.VMEM((2, KK, PCH, D), jnp.float32),
            pltpu.VMEM((2, PCH, D), jnp.float32),
            pltpu.SemaphoreType.DMA,
            pltpu.SemaphoreType.DMA,
            pltpu.SemaphoreType.DMA,
            pltpu.SemaphoreType.DMA,
            pltpu.SemaphoreType.DMA,
        ],
        compiler_params=pltpu.CompilerParams(use_tc_tiling_on_sc=False,
                                             needs_layout_passes=False),
    )
    def sc_combine(vt, idxm, wm, out, idx_v, w_v, rows_v, ob_v,
                   sem_i, sem_w, sem0, sem1, sem_o):
        wid = lax.axis_index("s") * NC + lax.axis_index("c")
        base = wid * PPW
        for j in range(KK):
            pltpu.async_copy(idxm.at[j, pl.ds(base, PPW)], idx_v.at[j], sem_i)
            pltpu.async_copy(wm.at[j, pl.ds(base, PPW)], w_v.at[j], sem_w)
        for j in range(KK):
            pltpu.make_async_copy(idxm.at[j, pl.ds(base, PPW)],
                                  idx_v.at[j], sem_i).wait()
        sems = (sem0, sem1)

        def issue(ci, b):
            p0 = ci * PCH
            for j in range(KK):
                pltpu.async_copy(vt.at[idx_v[j, pl.ds(p0, PCH)]],
                                 rows_v.at[b, j], sems[b])

        def drain_rows(b):
            for j in range(KK):
                pltpu.make_async_copy(vt.at[pl.ds(0, PCH)],
                                      rows_v.at[b, j], sems[b]).wait()

        issue(0, 0)
        for j in range(KK):
            pltpu.make_async_copy(wm.at[j, pl.ds(base, PPW)],
                                  w_v.at[j], sem_w).wait()

        @pl.loop(0, NCH, step=2)
        def _chunkpair(ci):
            for b in range(2):
                cur = ci + b
                nxt = cur + 1

                @pl.when(nxt < NCH)
                def _():
                    issue(nxt, 1 - b)

                drain_rows(b)
                @pl.when(cur >= 2)
                def _():
                    pltpu.make_async_copy(ob_v.at[b],
                                          out.at[pl.ds(base, PCH)],
                                          sem_o).wait()
                p0 = cur * PCH
                wv = [w_v[j, pl.ds(p0, PCH)] for j in range(KK)]
                for p in range(PCH):
                    pv = jnp.full((PCH, 1), p, jnp.int32)
                    wb = [_bcast_lane(wv[j], pv) for j in range(KK)]

                    @pl.loop(0, D // PCH, unroll=4)
                    def _cc(cc):
                        c0 = cc * PCH
                        acc0 = wb[0] * rows_v[b, 0, p, pl.ds(c0, PCH)]
                        acc1 = wb[1] * rows_v[b, 1, p, pl.ds(c0, PCH)]
                        acc2 = wb[2] * rows_v[b, 2, p, pl.ds(c0, PCH)]
                        acc3 = wb[3] * rows_v[b, 3, p, pl.ds(c0, PCH)]
                        acc0 = acc0 + wb[4] * rows_v[b, 4, p, pl.ds(c0, PCH)]
                        acc1 = acc1 + wb[5] * rows_v[b, 5, p, pl.ds(c0, PCH)]
                        acc2 = acc2 + wb[6] * rows_v[b, 6, p, pl.ds(c0, PCH)]
                        acc3 = acc3 + wb[7] * rows_v[b, 7, p, pl.ds(c0, PCH)]
                        ob_v[b, p, pl.ds(c0, PCH)] = (acc0 + acc1) + (acc2 + acc3)

                pltpu.async_copy(ob_v.at[b], out.at[pl.ds(base + p0, PCH)],
                                 sem_o)
        for b in range(2):
            pltpu.make_async_copy(ob_v.at[b], out.at[pl.ds(base, PCH)],
                                  sem_o).wait()

    return sc_combine


def kernel(q, k, v):
    qf = q[0].reshape(C, N)
    kf = k[0].reshape(C, N)
    idxm, wm = _topk_call(qf, kf)
    vt = v.reshape(B, C, N).transpose(2, 0, 1).reshape(N, D)
    out = _make_sc_combine()(vt, idxm, wm)
    return out.reshape(64, 64, B, C).transpose(2, 3, 0, 1)


# speedup vs baseline: 10.0388x; 1.1123x over previous
import functools

import jax
import jax.numpy as jnp
from jax import lax
from jax.experimental import pallas as pl
from jax.experimental.pallas import tpu as pltpu
from jax.experimental.pallas import tpu_sc as plsc

N = 4096
C = 64
B = 4
KK = 8
QB = 512
D = B * C
NSLICES = 2
NSL = N // NSLICES
NC, NS = 2, 16
NW = NC * NS
PPW = NSL // NW
PCH = 16
NCH = PPW // PCH


def _vt_body(v_ref, vt_ref):
    vt_ref[...] = v_ref[...].reshape(D, QB).T


def _vt_call(v):
    return pl.pallas_call(
        _vt_body,
        grid=(N // QB,),
        in_specs=[pl.BlockSpec((B, C, QB // 64, 64), lambda i: (0, 0, i, 0))],
        out_specs=pl.BlockSpec((QB, D), lambda i: (i, 0)),
        out_shape=jax.ShapeDtypeStruct((N, D), jnp.float32),
    )(v)


def _topk_body(qb_ref, kf_ref, idx_ref, w_ref):
    qb = qb_ref[...].reshape(C, QB)
    kf = kf_ref[...].reshape(C, N)
    s = lax.dot_general(qb, kf, (((0,), (0,)), ((), ())),
                        preferred_element_type=jnp.float32)
    kn = jnp.sum(kf * kf, axis=0)
    cur = kn[None, :] - 2.0 * s
    fiota = lax.broadcasted_iota(jnp.int32, (QB, N), 1).astype(jnp.float32)
    vals, idxs = [], []
    for j in range(KK):
        m = jnp.min(cur, axis=1, keepdims=True)
        im = jnp.where(cur <= m, fiota, jnp.float32(1.0e9))
        mi = jnp.min(im, axis=1, keepdims=True)
        vals.append(m)
        idxs.append(mi)
        if j + 1 < KK:
            cur = jnp.where(fiota == mi, jnp.float32(3.0e38), cur)
    es = [jnp.exp(vals[0] - vj) for vj in vals]
    den = es[0]
    for e in es[1:]:
        den = den + e
    for j in range(KK):
        idx_ref[j, :] = idxs[j][:, 0].astype(jnp.int32)
        w_ref[j, :] = (es[j] / den)[:, 0]


def _topk_call(q, k, off):
    qbw = QB // 64
    return pl.pallas_call(
        _topk_body,
        grid=(NSL // QB,),
        in_specs=[pl.BlockSpec((1, C, qbw, 64),
                               lambda i, off=off: (0, 0, i + off, 0)),
                  pl.BlockSpec((1, C, 64, 64), lambda i: (0, 0, 0, 0))],
        out_specs=[pl.BlockSpec((KK, QB), lambda i: (0, i)),
                   pl.BlockSpec((KK, QB), lambda i: (0, i))],
        out_shape=[jax.ShapeDtypeStruct((KK, NSL), jnp.int32),
                   jax.ShapeDtypeStruct((KK, NSL), jnp.float32)],
    )(q, k)


def _bcast_lane(vec, idx):
    dn = lax.GatherDimensionNumbers(offset_dims=(), collapsed_slice_dims=(0,),
                                    start_index_map=(0,))
    return lax.gather(vec, idx, dn, (1,),
                      mode=lax.GatherScatterMode.PROMISE_IN_BOUNDS)


@functools.cache
def _make_sc_combine():
    mesh = plsc.VectorSubcoreMesh(core_axis_name="c", subcore_axis_name="s",
                                  num_cores=NC, num_subcores=NS)

    @functools.partial(
        pl.kernel,
        out_type=jax.ShapeDtypeStruct((NSL, D), jnp.float32),
        mesh=mesh,
        scratch_types=[
            pltpu.VMEM((KK, PPW), jnp.int32),
            pltpu.VMEM((KK, PPW), jnp.float32),
            pltpu.VMEM((2, KK, PCH, D), jnp.float32),
            pltpu.VMEM((2, PCH, D), jnp.float32),
            pltpu.SemaphoreType.DMA,
            pltpu.SemaphoreType.DMA,
            pltpu.SemaphoreType.DMA,
            pltpu.SemaphoreType.DMA,
            pltpu.SemaphoreType.DMA,
        ],
        compiler_params=pltpu.CompilerParams(use_tc_tiling_on_sc=False,
                                             needs_layout_passes=False),
    )
    def sc_combine(vt, idxm, wm, out, idx_v, w_v, rows_v, ob_v,
                   sem_i, sem_w, sem0, sem1, sem_o):
        wid = lax.axis_index("s") * NC + lax.axis_index("c")
        base = wid * PPW
        for j in range(KK):
            pltpu.async_copy(idxm.at[j, pl.ds(base, PPW)], idx_v.at[j], sem_i)
            pltpu.async_copy(wm.at[j, pl.ds(base, PPW)], w_v.at[j], sem_w)
        for j in range(KK):
            pltpu.make_async_copy(idxm.at[j, pl.ds(base, PPW)],
                                  idx_v.at[j], sem_i).wait()
        sems = (sem0, sem1)

        def issue(ci, b):
            p0 = ci * PCH
            for j in range(KK):
                pltpu.async_copy(vt.at[idx_v[j, pl.ds(p0, PCH)]],
                                 rows_v.at[b, j], sems[b])

        def drain_rows(b):
            for j in range(KK):
                pltpu.make_async_copy(vt.at[pl.ds(0, PCH)],
                                      rows_v.at[b, j], sems[b]).wait()

        issue(0, 0)
        for j in range(KK):
            pltpu.make_async_copy(wm.at[j, pl.ds(base, PPW)],
                                  w_v.at[j], sem_w).wait()

        @pl.loop(0, NCH, step=2)
        def _chunkpair(ci):
            for b in range(2):
                cur = ci + b
                nxt = cur + 1

                @pl.when(nxt < NCH)
                def _():
                    issue(nxt, 1 - b)

                drain_rows(b)
                @pl.when(cur >= 2)
                def _():
                    pltpu.make_async_copy(ob_v.at[b],
                                          out.at[pl.ds(base, PCH)],
                                          sem_o).wait()
                p0 = cur * PCH
                wv = [w_v[j, pl.ds(p0, PCH)] for j in range(KK)]
                for p in range(PCH):
                    pv = jnp.full((PCH, 1), p, jnp.int32)
                    wb = [_bcast_lane(wv[j], pv) for j in range(KK)]

                    @pl.loop(0, D // PCH, unroll=4)
                    def _cc(cc):
                        c0 = cc * PCH
                        acc0 = wb[0] * rows_v[b, 0, p, pl.ds(c0, PCH)]
                        acc1 = wb[1] * rows_v[b, 1, p, pl.ds(c0, PCH)]
                        acc2 = wb[2] * rows_v[b, 2, p, pl.ds(c0, PCH)]
                        acc3 = wb[3] * rows_v[b, 3, p, pl.ds(c0, PCH)]
                        acc0 = acc0 + wb[4] * rows_v[b, 4, p, pl.ds(c0, PCH)]
                        acc1 = acc1 + wb[5] * rows_v[b, 5, p, pl.ds(c0, PCH)]
                        acc2 = acc2 + wb[6] * rows_v[b, 6, p, pl.ds(c0, PCH)]
                        acc3 = acc3 + wb[7] * rows_v[b, 7, p, pl.ds(c0, PCH)]
                        ob_v[b, p, pl.ds(c0, PCH)] = (acc0 + acc1) + (acc2 + acc3)

                pltpu.async_copy(ob_v.at[b], out.at[pl.ds(base + p0, PCH)],
                                 sem_o)
        for b in range(2):
            pltpu.make_async_copy(ob_v.at[b], out.at[pl.ds(base, PCH)],
                                  sem_o).wait()

    return sc_combine


def _asm_body(a_ref, b_ref, t_ref):
    i = pl.program_id(0)
    half = NSL // QB
    x = jnp.where(i < half, a_ref[...], b_ref[...])
    t_ref[...] = x.T.reshape(B, C, QB // 64, 64)


def _assemble_call(o0, o1):
    half = NSL // QB
    return pl.pallas_call(
        _asm_body,
        grid=(N // QB,),
        in_specs=[pl.BlockSpec((QB, D),
                               lambda i: (jnp.minimum(i, half - 1), 0)),
                  pl.BlockSpec((QB, D),
                               lambda i: (jnp.maximum(i, half) - half, 0))],
        out_specs=pl.BlockSpec((B, C, QB // 64, 64), lambda i: (0, 0, i, 0)),
        out_shape=jax.ShapeDtypeStruct((B, C, 64, 64), jnp.float32),
    )(o0, o1)


def kernel(q, k, v):
    vt = _vt_call(v)
    sc = _make_sc_combine()
    outs = []
    for s in range(NSLICES):
        idxm, wm = _topk_call(q, k, s * (NSL // QB))
        outs.append(sc(vt, idxm, wm))
    return _assemble_call(*outs)


# speedup vs baseline: 10.5198x; 1.0479x over previous
import functools

import jax
import jax.numpy as jnp
from jax import lax
from jax.experimental import pallas as pl
from jax.experimental.pallas import tpu as pltpu
from jax.experimental.pallas import tpu_sc as plsc

N = 4096
C = 64
B = 4
KK = 8
QB = 512
D = B * C
NC, NS = 2, 16
NW = NC * NS
PPW = N // NW
PCH = 16
NCH = PPW // PCH


def _topk_body(qb_ref, kf_ref, vb_ref, idx_ref, w_ref, vt_ref):
    qb = qb_ref[...].reshape(C, QB)
    kf = kf_ref[...].reshape(C, N)
    vt_ref[...] = vb_ref[...].reshape(D, QB).T
    s = lax.dot_general(qb, kf, (((0,), (0,)), ((), ())),
                        preferred_element_type=jnp.float32)
    kn = jnp.sum(kf * kf, axis=0)
    cur = kn[None, :] - 2.0 * s
    iota = lax.broadcasted_iota(jnp.int32, (QB, N), 1)
    m1 = jnp.min(cur, axis=1, keepdims=True)
    zb = lax.bitcast_convert_type(cur - m1 + jnp.float32(1.0), jnp.int32)
    pkf = lax.bitcast_convert_type((zb & jnp.int32(~0xFFF)) | iota,
                                   jnp.float32)
    ones = jnp.ones((N, 1), jnp.float32)
    vals, idxs = [], []
    for j in range(KK):
        pmin = jnp.min(pkf, axis=1, keepdims=True)
        maskb = pkf == pmin
        valj = lax.dot_general(jnp.where(maskb, cur, jnp.float32(0.0)),
                               ones, (((1,), (0,)), ((), ())),
                               preferred_element_type=jnp.float32)
        idxs.append(lax.bitcast_convert_type(pmin, jnp.int32)
                    & jnp.int32(0xFFF))
        vals.append(valj)
        if j + 1 < KK:
            pkf = jnp.where(maskb, jnp.float32(3.0e38), pkf)
    vm = vals[0]
    for v_ in vals[1:]:
        vm = jnp.minimum(vm, v_)
    es = [jnp.exp(vm - vj) for vj in vals]
    den = es[0]
    for e in es[1:]:
        den = den + e
    for j in range(KK):
        idx_ref[j, :] = idxs[j][:, 0]
        w_ref[j, :] = (es[j] / den)[:, 0]


def _topk_call(q, k, v):
    qbw = QB // 64
    return pl.pallas_call(
        _topk_body,
        grid=(N // QB,),
        in_specs=[pl.BlockSpec((1, C, qbw, 64), lambda i: (0, 0, i, 0)),
                  pl.BlockSpec((1, C, 64, 64), lambda i: (0, 0, 0, 0)),
                  pl.BlockSpec((B, C, qbw, 64), lambda i: (0, 0, i, 0))],
        out_specs=[pl.BlockSpec((KK, QB), lambda i: (0, i)),
                   pl.BlockSpec((KK, QB), lambda i: (0, i)),
                   pl.BlockSpec((QB, D), lambda i: (i, 0))],
        out_shape=[jax.ShapeDtypeStruct((KK, N), jnp.int32),
                   jax.ShapeDtypeStruct((KK, N), jnp.float32),
                   jax.ShapeDtypeStruct((N, D), jnp.float32)],
    )(q, k, v)


def _bcast_lane(vec, idx):
    dn = lax.GatherDimensionNumbers(offset_dims=(), collapsed_slice_dims=(0,),
                                    start_index_map=(0,))
    return lax.gather(vec, idx, dn, (1,),
                      mode=lax.GatherScatterMode.PROMISE_IN_BOUNDS)


@functools.cache
def _make_sc_combine():
    mesh = plsc.VectorSubcoreMesh(core_axis_name="c", subcore_axis_name="s",
                                  num_cores=NC, num_subcores=NS)

    @functools.partial(
        pl.kernel,
        out_type=jax.ShapeDtypeStruct((N, D), jnp.float32),
        mesh=mesh,
        scratch_types=[
            pltpu.VMEM((KK, PPW), jnp.int32),
            pltpu.VMEM((KK, PPW), jnp.float32),
            pltpu.VMEM((2, KK, PCH, D), jnp.float32),
            pltpu.VMEM((2, PCH, D), jnp.float32),
            pltpu.SemaphoreType.DMA,
            pltpu.SemaphoreType.DMA,
            pltpu.SemaphoreType.DMA,
            pltpu.SemaphoreType.DMA,
            pltpu.SemaphoreType.DMA,
        ],
        compiler_params=pltpu.CompilerParams(use_tc_tiling_on_sc=False,
                                             needs_layout_passes=False),
    )
    def sc_combine(vt, idxm, wm, out, idx_v, w_v, rows_v, ob_v,
                   sem_i, sem_w, sem0, sem1, sem_o):
        wid = lax.axis_index("s") * NC + lax.axis_index("c")
        base = wid * PPW
        for j in range(KK):
            pltpu.async_copy(idxm.at[j, pl.ds(base, PPW)], idx_v.at[j], sem_i)
            pltpu.async_copy(wm.at[j, pl.ds(base, PPW)], w_v.at[j], sem_w)
        for j in range(KK):
            pltpu.make_async_copy(idxm.at[j, pl.ds(base, PPW)],
                                  idx_v.at[j], sem_i).wait()
        sems = (sem0, sem1)

        def issue(ci, b):
            p0 = ci * PCH
            for j in range(KK):
                pltpu.async_copy(vt.at[idx_v[j, pl.ds(p0, PCH)]],
                                 rows_v.at[b, j], sems[b])

        def drain_rows(b):
            for j in range(KK):
                pltpu.make_async_copy(vt.at[pl.ds(0, PCH)],
                                      rows_v.at[b, j], sems[b]).wait()

        issue(0, 0)
        for j in range(KK):
            pltpu.make_async_copy(wm.at[j, pl.ds(base, PPW)],
                                  w_v.at[j], sem_w).wait()

        @pl.loop(0, NCH, step=2)
        def _chunkpair(ci):
            for b in range(2):
                cur = ci + b
                nxt = cur + 1

                @pl.when(nxt < NCH)
                def _():
                    issue(nxt, 1 - b)

                drain_rows(b)
                @pl.when(cur >= 2)
                def _():
                    pltpu.make_async_copy(ob_v.at[b],
                                          out.at[pl.ds(base, PCH)],
                                          sem_o).wait()
                p0 = cur * PCH
                wv = [w_v[j, pl.ds(p0, PCH)] for j in range(KK)]
                for p in range(PCH):
                    pv = jnp.full((PCH, 1), p, jnp.int32)
                    wb = [_bcast_lane(wv[j], pv) for j in range(KK)]

                    @pl.loop(0, D // PCH, unroll=4)
                    def _cc(cc):
                        c0 = cc * PCH
                        acc0 = wb[0] * rows_v[b, 0, p, pl.ds(c0, PCH)]
                        acc1 = wb[1] * rows_v[b, 1, p, pl.ds(c0, PCH)]
                        acc2 = wb[2] * rows_v[b, 2, p, pl.ds(c0, PCH)]
                        acc3 = wb[3] * rows_v[b, 3, p, pl.ds(c0, PCH)]
                        acc0 = acc0 + wb[4] * rows_v[b, 4, p, pl.ds(c0, PCH)]
                        acc1 = acc1 + wb[5] * rows_v[b, 5, p, pl.ds(c0, PCH)]
                        acc2 = acc2 + wb[6] * rows_v[b, 6, p, pl.ds(c0, PCH)]
                        acc3 = acc3 + wb[7] * rows_v[b, 7, p, pl.ds(c0, PCH)]
                        ob_v[b, p, pl.ds(c0, PCH)] = (acc0 + acc1) + (acc2 + acc3)

                pltpu.async_copy(ob_v.at[b], out.at[pl.ds(base + p0, PCH)],
                                 sem_o)
        for b in range(2):
            pltpu.make_async_copy(ob_v.at[b], out.at[pl.ds(base, PCH)],
                                  sem_o).wait()

    return sc_combine


def _asm_body(o_ref, t_ref):
    t_ref[...] = o_ref[...].T.reshape(B, C, QB // 64, 64)


def _assemble_call(o):
    return pl.pallas_call(
        _asm_body,
        grid=(N // QB,),
        in_specs=[pl.BlockSpec((QB, D), lambda i: (i, 0))],
        out_specs=pl.BlockSpec((B, C, QB // 64, 64), lambda i: (0, 0, i, 0)),
        out_shape=jax.ShapeDtypeStruct((B, C, 64, 64), jnp.float32),
    )(o)


def kernel(q, k, v):
    idxm, wm, vt = _topk_call(q, k, v)
    out = _make_sc_combine()(vt, idxm, wm)
    return _assemble_call(out)
